# initial kernel scaffold (unmeasured)
import jax
import jax.numpy as jnp
from jax import lax
from jax.experimental import pallas as pl
from jax.experimental.pallas import tpu as pltpu

N_DEV = 4


def kernel(x, w_mat):
    m_global, _ = x.shape
    _, n = w_mat.shape
    m_per = m_global // N_DEV

    def body(
        x_ref,
        w_ref,
        out_ref,
        send_ref,
        recv_ref,
        amax_ref,
        send_sems,
        recv_sems,
        amax_send_sems,
        amax_recv_sems,
    ):
        my = lax.axis_index("i")
        right = lax.rem(my + 1, N_DEV)

        w_bf = w_ref[...].astype(jnp.bfloat16)

        def partial(j):
            xc = x_ref[pl.ds(j * m_per, m_per), :].astype(jnp.bfloat16)
            return jnp.dot(xc, w_bf, preferred_element_type=jnp.float32)

        rdmas = []
        for s in range(N_DEV - 1):
            j = lax.rem(my - (s + 1) + N_DEV, N_DEV)
            part = partial(j)
            if s == 0:
                acc = part
            else:
                rdmas[s - 1].wait_recv()
                acc = recv_ref[s - 1].astype(jnp.float32) + part
            send_ref[s] = acc.astype(jnp.bfloat16)
            rdma = pltpu.make_async_remote_copy(
                src_ref=send_ref.at[s],
                dst_ref=recv_ref.at[s],
                send_sem=send_sems.at[s],
                recv_sem=recv_sems.at[s],
                device_id=(right,),
                device_id_type=pl.DeviceIdType.MESH,
            )
            rdma.start()
            rdmas.append(rdma)

        part = partial(my)
        rdmas[-1].wait_recv()
        y = recv_ref[N_DEV - 2].astype(jnp.float32) + part
        y = jnp.maximum(y, 0.0)

        amax_ref[0] = jnp.full((8, 128), jnp.max(y), jnp.float32)
        amax_rdmas = []
        for d in range(1, N_DEV):
            peer = lax.rem(my + d, N_DEV)
            r = pltpu.make_async_remote_copy(
                src_ref=amax_ref.at[0],
                dst_ref=amax_ref.at[d],
                send_sem=amax_send_sems.at[d - 1],
                recv_sem=amax_recv_sems.at[d - 1],
                device_id=(peer,),
                device_id_type=pl.DeviceIdType.MESH,
            )
            r.start()
            amax_rdmas.append(r)
        for r in amax_rdmas:
            r.wait_recv()

        gmax = jnp.max(amax_ref[...])
        scale = gmax / 127.0
        q = jnp.clip(jnp.round(y / scale), 0.0, 127.0)
        out_ref[...] = q * scale

        for r in rdmas:
            r.wait_send()
        for r in amax_rdmas:
            r.wait_send()

    return pl.pallas_call(
        body,
        out_shape=jax.ShapeDtypeStruct((m_per, n), jnp.float32),
        in_specs=[
            pl.BlockSpec(memory_space=pltpu.VMEM),
            pl.BlockSpec(memory_space=pltpu.VMEM),
        ],
        out_specs=pl.BlockSpec(memory_space=pltpu.VMEM),
        scratch_shapes=[
            pltpu.VMEM((N_DEV - 1, m_per, n), jnp.bfloat16),
            pltpu.VMEM((N_DEV - 1, m_per, n), jnp.bfloat16),
            pltpu.VMEM((N_DEV, 8, 128), jnp.float32),
            pltpu.SemaphoreType.DMA((N_DEV - 1,)),
            pltpu.SemaphoreType.DMA((N_DEV - 1,)),
            pltpu.SemaphoreType.DMA((N_DEV - 1,)),
            pltpu.SemaphoreType.DMA((N_DEV - 1,)),
        ],
        compiler_params=pltpu.CompilerParams(collective_id=0),
    )(x, w_mat)


# baseline (device time: 178582 ns/iter reference)
import jax
import jax.numpy as jnp
from jax import lax
from jax.experimental import pallas as pl
from jax.experimental.pallas import tpu as pltpu

N_DEV = 4


def kernel(x, w_mat):
    m_global, _ = x.shape
    _, n = w_mat.shape
    m_per = m_global // N_DEV

    def body(
        x_ref,
        w_ref,
        out_ref,
        send_ref,
        recv_ref,
        amax_ref,
        send_sems,
        recv_sems,
        amax_send_sems,
        amax_recv_sems,
    ):
        my = lax.axis_index("i")
        right = lax.rem(my + 1, N_DEV)

        w_bf = w_ref[...].astype(jnp.bfloat16)

        def partial(j):
            xc = x_ref[pl.ds(j * m_per, m_per), :].astype(jnp.bfloat16)
            return jnp.dot(xc, w_bf, preferred_element_type=jnp.float32)

        rdmas = []
        for s in range(N_DEV - 1):
            j = lax.rem(my - (s + 1) + N_DEV, N_DEV)
            part = partial(j)
            if s == 0:
                send_ref[...] = part.astype(jnp.bfloat16)
                src = send_ref
            else:
                rdmas[s - 1].wait_recv()
                acc = recv_ref[s - 1].astype(jnp.float32) + part
                recv_ref[s - 1] = acc.astype(jnp.bfloat16)
                src = recv_ref.at[s - 1]
            rdma = pltpu.make_async_remote_copy(
                src_ref=src,
                dst_ref=recv_ref.at[s],
                send_sem=send_sems.at[s],
                recv_sem=recv_sems.at[s],
                device_id=(right,),
                device_id_type=pl.DeviceIdType.MESH,
            )
            rdma.start()
            rdmas.append(rdma)

        part = partial(my)
        rdmas[-1].wait_recv()
        y = recv_ref[N_DEV - 2].astype(jnp.float32) + part
        y = jnp.maximum(y, 0.0)
        out_ref[...] = y

        amax_ref[0] = jnp.full((8, 128), jnp.max(y), jnp.float32)
        amax_rdmas = []
        for d in range(1, N_DEV):
            peer = lax.rem(my + d, N_DEV)
            r = pltpu.make_async_remote_copy(
                src_ref=amax_ref.at[0],
                dst_ref=amax_ref.at[d],
                send_sem=amax_send_sems.at[d - 1],
                recv_sem=amax_recv_sems.at[d - 1],
                device_id=(peer,),
                device_id_type=pl.DeviceIdType.MESH,
            )
            r.start()
            amax_rdmas.append(r)
        for r in amax_rdmas:
            r.wait_recv()

        gmax = jnp.max(amax_ref[...])
        scale = gmax / 127.0
        q = jnp.clip(jnp.round(out_ref[...] / scale), 0.0, 127.0)
        out_ref[...] = q * scale

        for r in rdmas:
            r.wait_send()
        for r in amax_rdmas:
            r.wait_send()

    return pl.pallas_call(
        body,
        out_shape=jax.ShapeDtypeStruct((m_per, n), jnp.float32),
        in_specs=[
            pl.BlockSpec(memory_space=pltpu.VMEM),
            pl.BlockSpec(memory_space=pltpu.VMEM),
        ],
        out_specs=pl.BlockSpec(memory_space=pltpu.VMEM),
        scratch_shapes=[
            pltpu.VMEM((m_per, n), jnp.bfloat16),
            pltpu.VMEM((N_DEV - 1, m_per, n), jnp.bfloat16),
            pltpu.VMEM((N_DEV, 8, 128), jnp.float32),
            pltpu.SemaphoreType.DMA((N_DEV - 1,)),
            pltpu.SemaphoreType.DMA((N_DEV - 1,)),
            pltpu.SemaphoreType.DMA((N_DEV - 1,)),
            pltpu.SemaphoreType.DMA((N_DEV - 1,)),
        ],
        compiler_params=pltpu.CompilerParams(
            vmem_limit_bytes=128 * 1024 * 1024,
        ),
    )(x, w_mat)


# device time: 108817 ns/iter; 1.6411x vs baseline; 1.6411x over previous
import jax
import jax.numpy as jnp
from jax import lax
from jax.experimental import pallas as pl
from jax.experimental.pallas import tpu as pltpu

N_DEV = 4


def kernel(x, w_mat):
    m_global, _ = x.shape
    _, n = w_mat.shape
    m_per = m_global // N_DEV
    h = n // 2

    def body(
        x_ref,
        w_ref,
        out_ref,
        send_r_ref,
        send_l_ref,
        recv_r_ref,
        recv_l_ref,
        amax_ref,
        send_sems_r,
        recv_sems_r,
        send_sems_l,
        recv_sems_l,
        amax_send_sems,
        amax_recv_sems,
    ):
        my = lax.axis_index("i")
        right = lax.rem(my + 1, N_DEV)
        left = lax.rem(my - 1 + N_DEV, N_DEV)

        w_bf = w_ref[...].astype(jnp.bfloat16)

        def partial(j, lo):
            xc = x_ref[pl.ds(j * m_per, m_per), :].astype(jnp.bfloat16)
            return jnp.dot(
                xc, w_bf[:, lo : lo + h], preferred_element_type=jnp.float32
            )

        rdmas_r = []
        rdmas_l = []
        for s in range(N_DEV - 1):
            jr = lax.rem(my - (s + 1) + N_DEV, N_DEV)
            jl = lax.rem(my + s + 1, N_DEV)
            part_r = partial(jr, 0)
            part_l = partial(jl, h)
            if s == 0:
                send_r_ref[...] = part_r.astype(jnp.bfloat16)
                src_r = send_r_ref
            else:
                rdmas_r[s - 1].wait_recv()
                acc = recv_r_ref[s - 1].astype(jnp.float32) + part_r
                recv_r_ref[s - 1] = acc.astype(jnp.bfloat16)
                src_r = recv_r_ref.at[s - 1]
            rdma_r = pltpu.make_async_remote_copy(
                src_ref=src_r,
                dst_ref=recv_r_ref.at[s],
                send_sem=send_sems_r.at[s],
                recv_sem=recv_sems_r.at[s],
                device_id=(right,),
                device_id_type=pl.DeviceIdType.MESH,
            )
            rdma_r.start()
            rdmas_r.append(rdma_r)

            if s == 0:
                send_l_ref[...] = part_l.astype(jnp.bfloat16)
                src_l = send_l_ref
            else:
                rdmas_l[s - 1].wait_recv()
                acc = recv_l_ref[s - 1].astype(jnp.float32) + part_l
                recv_l_ref[s - 1] = acc.astype(jnp.bfloat16)
                src_l = recv_l_ref.at[s - 1]
            rdma_l = pltpu.make_async_remote_copy(
                src_ref=src_l,
                dst_ref=recv_l_ref.at[s],
                send_sem=send_sems_l.at[s],
                recv_sem=recv_sems_l.at[s],
                device_id=(left,),
                device_id_type=pl.DeviceIdType.MESH,
            )
            rdma_l.start()
            rdmas_l.append(rdma_l)

        part_r = partial(my, 0)
        part_l = partial(my, h)
        rdmas_r[-1].wait_recv()
        y_r = jnp.maximum(recv_r_ref[N_DEV - 2].astype(jnp.float32) + part_r, 0.0)
        out_ref[:, 0:h] = y_r
        rdmas_l[-1].wait_recv()
        y_l = jnp.maximum(recv_l_ref[N_DEV - 2].astype(jnp.float32) + part_l, 0.0)
        out_ref[:, h:n] = y_l

        amax = jnp.maximum(jnp.max(y_r), jnp.max(y_l))
        amax_ref[0] = jnp.full((8, 128), amax, jnp.float32)
        amax_rdmas = []
        for d in range(1, N_DEV):
            peer = lax.rem(my + d, N_DEV)
            r = pltpu.make_async_remote_copy(
                src_ref=amax_ref.at[0],
                dst_ref=amax_ref.at[d],
                send_sem=amax_send_sems.at[d - 1],
                recv_sem=amax_recv_sems.at[d - 1],
                device_id=(peer,),
                device_id_type=pl.DeviceIdType.MESH,
            )
            r.start()
            amax_rdmas.append(r)
        for r in amax_rdmas:
            r.wait_recv()

        gmax = jnp.max(amax_ref[...])
        scale = gmax / 127.0
        q = jnp.clip(jnp.round(out_ref[...] / scale), 0.0, 127.0)
        out_ref[...] = q * scale

        for r in rdmas_r + rdmas_l + amax_rdmas:
            r.wait_send()

    return pl.pallas_call(
        body,
        out_shape=jax.ShapeDtypeStruct((m_per, n), jnp.float32),
        in_specs=[
            pl.BlockSpec(memory_space=pltpu.VMEM),
            pl.BlockSpec(memory_space=pltpu.VMEM),
        ],
        out_specs=pl.BlockSpec(memory_space=pltpu.VMEM),
        scratch_shapes=[
            pltpu.VMEM((m_per, h), jnp.bfloat16),
            pltpu.VMEM((m_per, h), jnp.bfloat16),
            pltpu.VMEM((N_DEV - 1, m_per, h), jnp.bfloat16),
            pltpu.VMEM((N_DEV - 1, m_per, h), jnp.bfloat16),
            pltpu.VMEM((N_DEV, 8, 128), jnp.float32),
            pltpu.SemaphoreType.DMA((N_DEV - 1,)),
            pltpu.SemaphoreType.DMA((N_DEV - 1,)),
            pltpu.SemaphoreType.DMA((N_DEV - 1,)),
            pltpu.SemaphoreType.DMA((N_DEV - 1,)),
            pltpu.SemaphoreType.DMA((N_DEV - 1,)),
            pltpu.SemaphoreType.DMA((N_DEV - 1,)),
        ],
        compiler_params=pltpu.CompilerParams(
            vmem_limit_bytes=128 * 1024 * 1024,
        ),
    )(x, w_mat)


# device time: 98541 ns/iter; 1.8123x vs baseline; 1.1043x over previous
import jax
import jax.numpy as jnp
from jax import lax
from jax.experimental import pallas as pl
from jax.experimental.pallas import tpu as pltpu

N_DEV = 4
N_RINGS = 4


def kernel(x, w_mat):
    m_global, _ = x.shape
    _, n = w_mat.shape
    m_per = m_global // N_DEV
    hh = n // N_RINGS
    h = n // 2

    rings = [
        dict(idx=0, dir_right=True, lo=0),
        dict(idx=1, dir_right=False, lo=h),
        dict(idx=2, dir_right=True, lo=hh),
        dict(idx=3, dir_right=False, lo=h + hh),
    ]

    def body(
        x_ref,
        w_ref,
        out_ref,
        send0_ref,
        recv_ref,
        amax_ref,
        send_sems,
        recv_sems,
        amax_send_sems,
        amax_recv_sems,
    ):
        my = lax.axis_index("i")
        right = lax.rem(my + 1, N_DEV)
        left = lax.rem(my - 1 + N_DEV, N_DEV)

        barrier_sem = pltpu.get_barrier_semaphore()
        for nbr in (left, right):
            pl.semaphore_signal(
                barrier_sem,
                inc=1,
                device_id=(nbr,),
                device_id_type=pl.DeviceIdType.MESH,
            )
        pl.semaphore_wait(barrier_sem, 2)

        w_bf = w_ref[...].astype(jnp.bfloat16)

        def sub_partial(j, lo):
            xc = x_ref[pl.ds(j * m_per, m_per), :].astype(jnp.bfloat16)
            return jnp.dot(
                xc, w_bf[:, lo : lo + hh], preferred_element_type=jnp.float32
            )

        def chunk_at(ring, s):
            if ring["dir_right"]:
                return lax.rem(my - (s + 1) + N_DEV, N_DEV)
            return lax.rem(my + s + 1, N_DEV)

        rdmas = {r["idx"]: [] for r in rings}
        for s in range(N_DEV - 1):
            for ring in rings:
                r = ring["idx"]
                part = sub_partial(chunk_at(ring, s), ring["lo"])
                if s == 0:
                    send0_ref[r] = part.astype(jnp.bfloat16)
                    src = send0_ref.at[r]
                else:
                    rdmas[r][s - 1].wait_recv()
                    acc = recv_ref[r, s - 1].astype(jnp.float32) + part
                    recv_ref[r, s - 1] = acc.astype(jnp.bfloat16)
                    src = recv_ref.at[r, s - 1]
                rdma = pltpu.make_async_remote_copy(
                    src_ref=src,
                    dst_ref=recv_ref.at[r, s],
                    send_sem=send_sems.at[r, s],
                    recv_sem=recv_sems.at[r, s],
                    device_id=(right if ring["dir_right"] else left,),
                    device_id_type=pl.DeviceIdType.MESH,
                )
                rdma.start()
                rdmas[r].append(rdma)

        amax = jnp.float32(0.0)
        for ring in rings:
            r = ring["idx"]
            lo = ring["lo"]
            part = sub_partial(my, lo)
            rdmas[r][-1].wait_recv()
            y = jnp.maximum(recv_ref[r, N_DEV - 2].astype(jnp.float32) + part, 0.0)
            out_ref[:, lo : lo + hh] = y
            amax = jnp.maximum(amax, jnp.max(y))

        amax_ref[0] = jnp.full((8, 128), amax, jnp.float32)
        amax_rdmas = []
        for d in range(1, N_DEV):
            peer = lax.rem(my + d, N_DEV)
            rr = pltpu.make_async_remote_copy(
                src_ref=amax_ref.at[0],
                dst_ref=amax_ref.at[d],
                send_sem=amax_send_sems.at[d - 1],
                recv_sem=amax_recv_sems.at[d - 1],
                device_id=(peer,),
                device_id_type=pl.DeviceIdType.MESH,
            )
            rr.start()
            amax_rdmas.append(rr)
        for rr in amax_rdmas:
            rr.wait_recv()

        gmax = jnp.max(amax_ref[...])
        scale = gmax / 127.0
        q = jnp.clip(jnp.round(out_ref[...] / scale), 0.0, 127.0)
        out_ref[...] = q * scale

        for r in rdmas:
            for rd in rdmas[r]:
                rd.wait_send()
        for rr in amax_rdmas:
            rr.wait_send()

    return pl.pallas_call(
        body,
        out_shape=jax.ShapeDtypeStruct((m_per, n), jnp.float32),
        in_specs=[
            pl.BlockSpec(memory_space=pltpu.VMEM),
            pl.BlockSpec(memory_space=pltpu.VMEM),
        ],
        out_specs=pl.BlockSpec(memory_space=pltpu.VMEM),
        scratch_shapes=[
            pltpu.VMEM((N_RINGS, m_per, hh), jnp.bfloat16),
            pltpu.VMEM((N_RINGS, N_DEV - 1, m_per, hh), jnp.bfloat16),
            pltpu.VMEM((N_DEV, 8, 128), jnp.float32),
            pltpu.SemaphoreType.DMA((N_RINGS, N_DEV - 1)),
            pltpu.SemaphoreType.DMA((N_RINGS, N_DEV - 1)),
            pltpu.SemaphoreType.DMA((N_DEV - 1,)),
            pltpu.SemaphoreType.DMA((N_DEV - 1,)),
        ],
        compiler_params=pltpu.CompilerParams(
            vmem_limit_bytes=128 * 1024 * 1024,
            collective_id=0,
        ),
    )(x, w_mat)


# device time: 98208 ns/iter; 1.8184x vs baseline; 1.0034x over previous
import jax
import jax.numpy as jnp
from jax import lax
from jax.experimental import pallas as pl
from jax.experimental.pallas import tpu as pltpu

N_DEV = 4
N_RINGS = 8


def kernel(x, w_mat):
    m_global, _ = x.shape
    _, n = w_mat.shape
    m_per = m_global // N_DEV
    hh = n // N_RINGS
    h = n // 2

    rings = []
    for k in range(N_RINGS // 2):
        rings.append(dict(idx=2 * k, dir_right=True, lo=k * hh))
        rings.append(dict(idx=2 * k + 1, dir_right=False, lo=h + k * hh))

    def body(
        x_ref,
        w_ref,
        out_ref,
        send0_ref,
        recv_ref,
        amax_ref,
        send_sems,
        recv_sems,
        amax_send_sems,
        amax_recv_sems,
    ):
        my = lax.axis_index("i")
        right = lax.rem(my + 1, N_DEV)
        left = lax.rem(my - 1 + N_DEV, N_DEV)

        barrier_sem = pltpu.get_barrier_semaphore()
        for nbr in (left, right):
            pl.semaphore_signal(
                barrier_sem,
                inc=1,
                device_id=(nbr,),
                device_id_type=pl.DeviceIdType.MESH,
            )
        pl.semaphore_wait(barrier_sem, 2)

        w_bf = w_ref[...].astype(jnp.bfloat16)

        def sub_partial(j, lo):
            xc = x_ref[pl.ds(j * m_per, m_per), :].astype(jnp.bfloat16)
            return jnp.dot(
                xc, w_bf[:, lo : lo + hh], preferred_element_type=jnp.float32
            )

        def chunk_at(ring, s):
            if ring["dir_right"]:
                return lax.rem(my - (s + 1) + N_DEV, N_DEV)
            return lax.rem(my + s + 1, N_DEV)

        rdmas = {r["idx"]: [] for r in rings}
        for s in range(N_DEV - 1):
            for ring in rings:
                r = ring["idx"]
                part = sub_partial(chunk_at(ring, s), ring["lo"])
                if s == 0:
                    send0_ref[r] = part.astype(jnp.bfloat16)
                    src = send0_ref.at[r]
                else:
                    rdmas[r][s - 1].wait_recv()
                    acc = recv_ref[r, s - 1].astype(jnp.float32) + part
                    recv_ref[r, s - 1] = acc.astype(jnp.bfloat16)
                    src = recv_ref.at[r, s - 1]
                rdma = pltpu.make_async_remote_copy(
                    src_ref=src,
                    dst_ref=recv_ref.at[r, s],
                    send_sem=send_sems.at[r, s],
                    recv_sem=recv_sems.at[r, s],
                    device_id=(right if ring["dir_right"] else left,),
                    device_id_type=pl.DeviceIdType.MESH,
                )
                rdma.start()
                rdmas[r].append(rdma)

        amax = jnp.float32(0.0)
        for ring in rings:
            r = ring["idx"]
            lo = ring["lo"]
            part = sub_partial(my, lo)
            rdmas[r][-1].wait_recv()
            y = jnp.maximum(recv_ref[r, N_DEV - 2].astype(jnp.float32) + part, 0.0)
            out_ref[:, lo : lo + hh] = y
            amax = jnp.maximum(amax, jnp.max(y))

        amax_ref[0] = jnp.full((8, 128), amax, jnp.float32)
        amax_rdmas = []
        for d in range(1, N_DEV):
            peer = lax.rem(my + d, N_DEV)
            rr = pltpu.make_async_remote_copy(
                src_ref=amax_ref.at[0],
                dst_ref=amax_ref.at[d],
                send_sem=amax_send_sems.at[d - 1],
                recv_sem=amax_recv_sems.at[d - 1],
                device_id=(peer,),
                device_id_type=pl.DeviceIdType.MESH,
            )
            rr.start()
            amax_rdmas.append(rr)
        for rr in amax_rdmas:
            rr.wait_recv()

        gmax = jnp.max(amax_ref[...])
        scale = gmax / 127.0
        q = jnp.clip(jnp.round(out_ref[...] / scale), 0.0, 127.0)
        out_ref[...] = q * scale

        for r in rdmas:
            for rd in rdmas[r]:
                rd.wait_send()
        for rr in amax_rdmas:
            rr.wait_send()

    return pl.pallas_call(
        body,
        out_shape=jax.ShapeDtypeStruct((m_per, n), jnp.float32),
        in_specs=[
            pl.BlockSpec(memory_space=pltpu.VMEM),
            pl.BlockSpec(memory_space=pltpu.VMEM),
        ],
        out_specs=pl.BlockSpec(memory_space=pltpu.VMEM),
        scratch_shapes=[
            pltpu.VMEM((N_RINGS, m_per, hh), jnp.bfloat16),
            pltpu.VMEM((N_RINGS, N_DEV - 1, m_per, hh), jnp.bfloat16),
            pltpu.VMEM((N_DEV, 8, 128), jnp.float32),
            pltpu.SemaphoreType.DMA((N_RINGS, N_DEV - 1)),
            pltpu.SemaphoreType.DMA((N_RINGS, N_DEV - 1)),
            pltpu.SemaphoreType.DMA((N_DEV - 1,)),
            pltpu.SemaphoreType.DMA((N_DEV - 1,)),
        ],
        compiler_params=pltpu.CompilerParams(
            vmem_limit_bytes=128 * 1024 * 1024,
            collective_id=0,
        ),
    )(x, w_mat)
